# baseline (speedup 1.0000x reference)
import functools
import math

import jax
import jax.numpy as jnp
import numpy as np
from jax import lax
from jax.experimental import pallas as pl
from jax.experimental.pallas import tpu as pltpu
from jax.experimental.pallas import tpu_sc as plsc

VOCAB = 100000
D_MODEL = 1024
BATCH = 4
SEQ = 8192

NUM_CORES = 2
NUM_SUBCORES = 16
NUM_WORKERS = NUM_CORES * NUM_SUBCORES
S_PER_WORKER = SEQ // NUM_WORKERS
CHUNK = 16
N_CHUNKS = S_PER_WORKER // CHUNK
N_ITEMS = N_CHUNKS * BATCH
LANES = 16
VECS_PER_ROW = D_MODEL // LANES
NBUF = 4


def _pe_scaled():
    pos = np.arange(SEQ, dtype=np.float32)[:, None]
    div = np.exp(
        np.arange(0, D_MODEL, 2, dtype=np.float32)
        * (-math.log(10000.0) / D_MODEL)
    )
    pe = np.zeros((SEQ, D_MODEL), dtype=np.float32)
    pe[:, 0::2] = np.sin(pos * div)
    pe[:, 1::2] = np.cos(pos * div)
    return np.asarray(pe * math.sqrt(D_MODEL), dtype=np.float32)


_PE32 = _pe_scaled()
_SCALE = math.sqrt(D_MODEL)


@functools.partial(
    pl.kernel,
    out_type=jax.ShapeDtypeStruct((BATCH, SEQ, D_MODEL), jnp.float32),
    mesh=plsc.VectorSubcoreMesh(core_axis_name="c", subcore_axis_name="s"),
    scratch_types=[
        pltpu.VMEM((BATCH, S_PER_WORKER), jnp.int32),
        pltpu.VMEM((2, CHUNK, D_MODEL), jnp.float32),
        pltpu.VMEM((CHUNK, D_MODEL), jnp.float32),
        pltpu.VMEM((CHUNK, D_MODEL), jnp.float32),
        pltpu.VMEM((CHUNK, D_MODEL), jnp.float32),
        pltpu.VMEM((CHUNK, D_MODEL), jnp.float32),
        pltpu.SemaphoreType.DMA,
        pltpu.SemaphoreType.DMA,
        pltpu.SemaphoreType.DMA,
        pltpu.SemaphoreType.DMA,
        pltpu.SemaphoreType.DMA,
        pltpu.SemaphoreType.DMA,
        pltpu.SemaphoreType.DMA,
        pltpu.SemaphoreType.DMA,
        pltpu.SemaphoreType.DMA,
    ],
)
def _emb_lookup(x_hbm, pe_hbm, table_hbm, out_hbm,
                idx_all, pe_v, rows0, rows1, rows2, rows3,
                g0, g1, g2, g3, w0, w1, w2, w3, psem):
    wid = lax.axis_index("s") * NUM_CORES + lax.axis_index("c")
    base = wid * S_PER_WORKER
    bufs = ((rows0, g0, w0), (rows1, g1, w1), (rows2, g2, w2), (rows3, g3, w3))

    def issue_gather(t, rows, gsem):
        idx = idx_all.at[t & 3, pl.ds((t >> 2) * CHUNK, CHUNK)]
        pltpu.async_copy(table_hbm.at[idx], rows, gsem)

    def out_view(t):
        return out_hbm.at[t & 3, pl.ds(base + (t >> 2) * CHUNK, CHUNK)]

    def issue_pe(c):
        pltpu.async_copy(
            pe_hbm.at[pl.ds(base + c * CHUNK, CHUNK)], pe_v.at[c & 1], psem
        )

    def wait_pe(c):
        pltpu.make_async_copy(
            pe_hbm.at[pl.ds(base, CHUNK)], pe_v.at[c & 1], psem
        ).wait()

    for b in range(BATCH):
        pltpu.sync_copy(x_hbm.at[b, pl.ds(base, S_PER_WORKER)], idx_all.at[b])
    issue_pe(0)
    for t in range(NBUF - 1):
        issue_gather(t, bufs[t][0], bufs[t][1])

    def step_fn(step, carry):
        for ph in range(NBUF):
            t = step * NBUF + ph
            rows, gsem, wsem = bufs[ph]
            c = t >> 2

            @pl.when((t & 3) == 0)
            def _():
                @pl.when(c + 1 < N_CHUNKS)
                def _():
                    issue_pe(c + 1)

                wait_pe(c)

            nt = t + NBUF - 1
            n_rows, n_gsem, n_wsem = bufs[(ph + NBUF - 1) % NBUF]

            @pl.when(jnp.logical_and(t >= 1, nt < N_ITEMS))
            def _():
                pltpu.make_async_copy(n_rows, out_view(t - 1), n_wsem).wait()

            @pl.when(nt < N_ITEMS)
            def _():
                issue_gather(nt, n_rows, n_gsem)

            pltpu.make_async_copy(
                table_hbm.at[idx_all.at[0, pl.ds(0, CHUNK)]], rows, gsem
            ).wait()

            cb = c & 1

            def row_fma(r, carry2):
                for j in range(VECS_PER_ROW):
                    sl = pl.ds(j * LANES, LANES)
                    rows[r, sl] = rows[r, sl] * _SCALE + pe_v[cb, r, sl]
                return carry2

            lax.fori_loop(0, CHUNK, row_fma, 0)

            pltpu.async_copy(rows, out_view(t), wsem)
        return carry

    lax.fori_loop(0, N_ITEMS // NBUF, step_fn, 0)

    for k in range(NBUF):
        t = N_ITEMS - NBUF + k
        pltpu.make_async_copy(bufs[t % NBUF][0], out_view(t),
                              bufs[t % NBUF][2]).wait()


def kernel(x, table):
    return _emb_lookup(x.astype(jnp.int32), jnp.asarray(_PE32), table)


# speedup vs baseline: 1.6363x; 1.6363x over previous
import functools
import math

import jax
import jax.numpy as jnp
import ml_dtypes
import numpy as np
from jax import lax
from jax.experimental import pallas as pl
from jax.experimental.pallas import tpu as pltpu
from jax.experimental.pallas import tpu_sc as plsc

VOCAB = 100000
D_MODEL = 1024
BATCH = 4
SEQ = 8192

NUM_CORES = 2
NUM_SUBCORES = 16
NUM_WORKERS = NUM_CORES * NUM_SUBCORES
S_PER_WORKER = SEQ // NUM_WORKERS
CHUNK = 32
N_CHUNKS = S_PER_WORKER // CHUNK
N_ITEMS = N_CHUNKS * BATCH
LANES = 16
WGROUPS = D_MODEL // (2 * LANES)
PE_WORDS = D_MODEL // 2
NBUF = 3


def _pe_packed():
    pos = np.arange(SEQ, dtype=np.float32)[:, None]
    div = np.exp(
        np.arange(0, D_MODEL, 2, dtype=np.float32)
        * (-math.log(10000.0) / D_MODEL)
    )
    pe = np.zeros((SEQ, D_MODEL), dtype=np.float32)
    pe[:, 0::2] = np.sin(pos * div)
    pe[:, 1::2] = np.cos(pos * div)
    pe *= math.sqrt(D_MODEL)
    u = pe.astype(ml_dtypes.bfloat16).view(np.uint16).astype(np.uint32)
    u = u.reshape(SEQ, WGROUPS, 2, LANES)
    words = (u[:, :, 1, :] << 16) | u[:, :, 0, :]
    return words.reshape(SEQ, PE_WORDS).view(np.int32)


_PE_PACKED = _pe_packed()
_SCALE = math.sqrt(D_MODEL)
_HI_MASK = np.int32(np.uint32(0xFFFF0000).view(np.int32))


@functools.partial(
    pl.kernel,
    out_type=jax.ShapeDtypeStruct((BATCH, SEQ, D_MODEL), jnp.float32),
    mesh=plsc.VectorSubcoreMesh(core_axis_name="c", subcore_axis_name="s"),
    scratch_types=[
        pltpu.VMEM((BATCH, S_PER_WORKER), jnp.int32),
        pltpu.VMEM((CHUNK, PE_WORDS), jnp.int32),
        pltpu.VMEM((CHUNK, D_MODEL), jnp.float32),
        pltpu.VMEM((CHUNK, D_MODEL), jnp.float32),
        pltpu.VMEM((CHUNK, D_MODEL), jnp.float32),
        pltpu.SemaphoreType.DMA,
        pltpu.SemaphoreType.DMA,
        pltpu.SemaphoreType.DMA,
        pltpu.SemaphoreType.DMA,
        pltpu.SemaphoreType.DMA,
        pltpu.SemaphoreType.DMA,
    ],
)
def _emb_lookup(x_hbm, pe_hbm, table_hbm, out_hbm,
                idx_all, pe_v, rows0, rows1, rows2,
                g0, g1, g2, w0, w1, w2):
    wid = lax.axis_index("s") * NUM_CORES + lax.axis_index("c")
    base = wid * S_PER_WORKER
    bufs = ((rows0, g0, w0), (rows1, g1, w1), (rows2, g2, w2))

    def issue_gather(t, rows, gsem):
        idx = idx_all.at[t & 3, pl.ds((t >> 2) * CHUNK, CHUNK)]
        pltpu.async_copy(table_hbm.at[idx], rows, gsem)

    def out_view(t):
        return out_hbm.at[t & 3, pl.ds(base + (t >> 2) * CHUNK, CHUNK)]

    for b in range(BATCH):
        pltpu.sync_copy(x_hbm.at[b, pl.ds(base, S_PER_WORKER)], idx_all.at[b])
    pltpu.sync_copy(pe_hbm.at[pl.ds(base, CHUNK)], pe_v)
    issue_gather(0, rows0, g0)

    def step_fn(step, carry):
        for ph in range(NBUF):
            t = step * NBUF + ph
            rows, gsem, wsem = bufs[ph]
            n_rows, n_gsem, n_wsem = bufs[(ph + 1) % NBUF]

            @pl.when(jnp.logical_and(t >= 2, t + 1 < N_ITEMS))
            def _():
                pltpu.make_async_copy(n_rows, out_view(t - 2), n_wsem).wait()

            @pl.when(t + 1 < N_ITEMS)
            def _():
                issue_gather(t + 1, n_rows, n_gsem)

            @pl.when(t < N_ITEMS)
            def _():
                pltpu.make_async_copy(
                    table_hbm.at[idx_all.at[0, pl.ds(0, CHUNK)]], rows, gsem
                ).wait()

                def row_fma(r, carry2):
                    for j in range(WGROUPS):
                        w = pe_v[r, pl.ds(j * LANES, LANES)]
                        lo = lax.bitcast_convert_type(
                            lax.shift_left(w, jnp.int32(16)), jnp.float32)
                        hi = lax.bitcast_convert_type(
                            lax.bitwise_and(w, _HI_MASK), jnp.float32)
                        sl_lo = pl.ds(j * 2 * LANES, LANES)
                        sl_hi = pl.ds(j * 2 * LANES + LANES, LANES)
                        rows[r, sl_lo] = rows[r, sl_lo] * _SCALE + lo
                        rows[r, sl_hi] = rows[r, sl_hi] * _SCALE + hi
                    return carry2

                lax.fori_loop(0, CHUNK, row_fma, 0)

                pltpu.async_copy(rows, out_view(t), wsem)

            @pl.when(jnp.logical_and((t & 3) == 3, t + 1 < N_ITEMS))
            def _():
                pltpu.sync_copy(
                    pe_hbm.at[pl.ds(base + ((t + 1) >> 2) * CHUNK, CHUNK)],
                    pe_v,
                )

        return carry

    lax.fori_loop(0, (N_ITEMS + NBUF) // NBUF, step_fn, 0)

    for t in range(N_ITEMS - NBUF, N_ITEMS):
        rows, _, wsem = bufs[t % NBUF]
        pltpu.make_async_copy(rows, out_view(t), wsem).wait()


def kernel(x, table):
    return _emb_lookup(x.astype(jnp.int32), jnp.asarray(_PE_PACKED), table)


# speedup vs baseline: 2.9707x; 1.8156x over previous
import functools
import math

import jax
import jax.numpy as jnp
import ml_dtypes
import numpy as np
from jax import lax
from jax.experimental import pallas as pl
from jax.experimental.pallas import tpu as pltpu
from jax.experimental.pallas import tpu_sc as plsc

VOCAB = 100000
D_MODEL = 1024
BATCH = 4
SEQ = 8192

NUM_CORES = 2
NUM_SUBCORES = 16
NUM_WORKERS = NUM_CORES * NUM_SUBCORES
S_PER_WORKER = SEQ // NUM_WORKERS
CHUNK = 32
N_CHUNKS = S_PER_WORKER // CHUNK
N_ITEMS = N_CHUNKS * BATCH
LANES = 16
WGROUPS = D_MODEL // (2 * LANES)
PE_WORDS = D_MODEL // 2
NBUF = 3


def _pe_packed():
    pos = np.arange(SEQ, dtype=np.float32)[:, None]
    div = np.exp(
        np.arange(0, D_MODEL, 2, dtype=np.float32)
        * (-math.log(10000.0) / D_MODEL)
    )
    pe = np.zeros((SEQ, D_MODEL), dtype=np.float32)
    pe[:, 0::2] = np.sin(pos * div)
    pe[:, 1::2] = np.cos(pos * div)
    pe *= math.sqrt(D_MODEL)
    u = pe.astype(ml_dtypes.bfloat16).view(np.uint16).astype(np.uint32)
    u = u.reshape(SEQ, WGROUPS, 2, LANES)
    words = (u[:, :, 1, :] << 16) | u[:, :, 0, :]
    return words.reshape(SEQ, PE_WORDS).view(np.int32)


_PE_PACKED = _pe_packed()
_SCALE = math.sqrt(D_MODEL)
_HI_MASK = np.int32(np.uint32(0xFFFF0000).view(np.int32))


@functools.partial(
    pl.kernel,
    out_type=jax.ShapeDtypeStruct((BATCH, SEQ, D_MODEL), jnp.float32),
    mesh=plsc.VectorSubcoreMesh(core_axis_name="c", subcore_axis_name="s"),
    scratch_types=[
        pltpu.VMEM((BATCH, S_PER_WORKER), jnp.int32),
        pltpu.VMEM((CHUNK, PE_WORDS), jnp.int32),
        pltpu.VMEM((CHUNK, D_MODEL), jnp.float32),
        pltpu.VMEM((CHUNK, D_MODEL), jnp.float32),
        pltpu.VMEM((CHUNK, D_MODEL), jnp.float32),
        pltpu.SemaphoreType.DMA,
        pltpu.SemaphoreType.DMA,
        pltpu.SemaphoreType.DMA,
        pltpu.SemaphoreType.DMA,
        pltpu.SemaphoreType.DMA,
        pltpu.SemaphoreType.DMA,
    ],
)
def _emb_lookup(x_hbm, pe_hbm, table_hbm, out_hbm,
                idx_all, pe_v, rows0, rows1, rows2,
                g0, g1, g2, w0, w1, w2):
    wid = lax.axis_index("s") * NUM_CORES + lax.axis_index("c")
    base = wid * S_PER_WORKER
    bufs = ((rows0, g0, w0), (rows1, g1, w1), (rows2, g2, w2))

    def issue_gather(t, rows, gsem):
        idx = idx_all.at[t & 3, pl.ds((t >> 2) * CHUNK, CHUNK)]
        pltpu.async_copy(table_hbm.at[idx], rows, gsem)

    def out_view(t):
        return out_hbm.at[t & 3, pl.ds(base + (t >> 2) * CHUNK, CHUNK)]

    for b in range(BATCH):
        pltpu.sync_copy(x_hbm.at[b, pl.ds(base, S_PER_WORKER)], idx_all.at[b])
    pltpu.sync_copy(pe_hbm.at[pl.ds(base, CHUNK)], pe_v)
    issue_gather(0, rows0, g0)

    def step_fn(step, carry):
        for ph in range(NBUF):
            t = step * NBUF + ph
            rows, gsem, wsem = bufs[ph]
            n_rows, n_gsem, n_wsem = bufs[(ph + 1) % NBUF]

            @pl.when(jnp.logical_and(t >= 2, t + 1 < N_ITEMS))
            def _():
                pltpu.make_async_copy(n_rows, out_view(t - 2), n_wsem).wait()

            @pl.when(t + 1 < N_ITEMS)
            def _():
                issue_gather(t + 1, n_rows, n_gsem)

            @pl.when(t < N_ITEMS)
            def _():
                pltpu.make_async_copy(
                    table_hbm.at[idx_all.at[0, pl.ds(0, CHUNK)]], rows, gsem
                ).wait()

                @plsc.parallel_loop(0, CHUNK * WGROUPS, 1, unroll=4)
                def _(g):
                    r = g >> 5
                    j = g & (WGROUPS - 1)
                    w = pe_v[r, pl.ds(j * LANES, LANES)]
                    lo = lax.bitcast_convert_type(
                        lax.shift_left(w, jnp.int32(16)), jnp.float32)
                    hi = lax.bitcast_convert_type(
                        lax.bitwise_and(w, _HI_MASK), jnp.float32)
                    sl_lo = pl.ds(j * 2 * LANES, LANES)
                    sl_hi = pl.ds(j * 2 * LANES + LANES, LANES)
                    rows[r, sl_lo] = rows[r, sl_lo] * _SCALE + lo
                    rows[r, sl_hi] = rows[r, sl_hi] * _SCALE + hi

                pltpu.async_copy(rows, out_view(t), wsem)

            @pl.when(jnp.logical_and((t & 3) == 3, t + 1 < N_ITEMS))
            def _():
                pltpu.sync_copy(
                    pe_hbm.at[pl.ds(base + ((t + 1) >> 2) * CHUNK, CHUNK)],
                    pe_v,
                )

        return carry

    lax.fori_loop(0, (N_ITEMS + NBUF) // NBUF, step_fn, 0)

    for t in range(N_ITEMS - NBUF, N_ITEMS):
        rows, _, wsem = bufs[t % NBUF]
        pltpu.make_async_copy(rows, out_view(t), wsem).wait()


def kernel(x, table):
    return _emb_lookup(x.astype(jnp.int32), jnp.asarray(_PE_PACKED), table)
